# baseline (speedup 1.0000x reference)
import jax
import jax.numpy as jnp
from jax.experimental import pallas as pl
from jax.experimental.pallas import tpu as pltpu

_B, _N, _D, _E, _C = 4, 2048, 4096, 64, 64
_BLKN = 512
_NB = _N // _BLKN
_EC = 8
_EB = _E // _EC


def _stats_body(x_ref, w_ref, meta_ref, aux_ref, zloss_ref,
                base_ref, counts_ref, dens_ref, zsq_ref):
    b = pl.program_id(0)
    nb = pl.program_id(1)

    @pl.when((b == 0) & (nb == 0))
    def _init_all():
        counts_ref[...] = jnp.zeros_like(counts_ref)
        dens_ref[...] = jnp.zeros_like(dens_ref)
        zsq_ref[...] = jnp.zeros_like(zsq_ref)

    @pl.when(nb == 0)
    def _init_batch():
        base_ref[...] = jnp.zeros_like(base_ref)

    x = x_ref[0]
    w = w_ref[...]
    lt = jax.lax.dot_general(w, x, (((0,), (1,)), ((), ())),
                             preferred_element_type=jnp.float32)
    m = jnp.max(lt, axis=0, keepdims=True)
    ex = jnp.exp(lt - m)
    s = jnp.sum(ex, axis=0, keepdims=True)
    z = m + jnp.log(s)
    zsq_ref[...] += jnp.sum(z * z).reshape(1, 1)
    gate = 1.0 / s
    dens_ref[...] += jnp.sum(ex / s, axis=1, keepdims=True)

    eids = jax.lax.broadcasted_iota(jnp.int32, (_E, _BLKN), 0)
    idx = jnp.min(jnp.where(lt == m, eids, _E), axis=0, keepdims=True)
    mask = (idx == eids).astype(jnp.float32)
    ri = jax.lax.broadcasted_iota(jnp.int32, (_BLKN, _BLKN), 0)
    ci = jax.lax.broadcasted_iota(jnp.int32, (_BLKN, _BLKN), 1)
    triu = (ri <= ci).astype(jnp.float32)
    cs = jnp.dot(mask, triu, preferred_element_type=jnp.float32)
    pos = base_ref[...] + cs
    tot = cs[:, _BLKN - 1:_BLKN]
    base_ref[...] += tot
    counts_ref[...] += tot
    p = jnp.sum(pos * mask, axis=0, keepdims=True) - 1.0

    meta_ref[0] = jnp.concatenate(
        [idx.astype(jnp.float32), p, gate,
         jnp.zeros((5, _BLKN), jnp.float32)], axis=0)

    @pl.when((b == _B - 1) & (nb == _NB - 1))
    def _fin():
        scale = 1.0 / (_B * _N)
        dens = dens_ref[...] * scale
        proxy = counts_ref[...] * scale
        aux_ref[...] = (jnp.sum(dens * proxy) * _E).reshape(1, 1)
        zloss_ref[...] = zsq_ref[...] * scale


def _writer_body(cap_ref, meta_ref, disp_ref, comb_ref):
    eb = pl.program_id(1)
    meta = meta_ref[0]
    idxr = meta[0:1, :].astype(jnp.int32)
    pr = meta[1:2, :].astype(jnp.int32)
    gate = meta[2:3, :]
    cap = cap_ref[0, 0]
    cids = jax.lax.broadcasted_iota(jnp.int32, (_C, _N), 0)
    ocf = ((pr == cids) & (cids < cap)).astype(jnp.float32)
    ocg = ocf * gate
    ech = jax.lax.broadcasted_iota(jnp.int32, (_EC, _N), 0) + eb * _EC
    mask = (idxr == ech).astype(jnp.float32)
    disp_ref[0] = mask[:, None, :] * ocf[None, :, :]
    comb_ref[0] = mask[:, None, :] * ocg[None, :, :]


def kernel(token_inputs, w_gate, expert_capacity):
    cap = jnp.asarray(expert_capacity, jnp.int32).reshape(1, 1)
    meta, aux, zloss = pl.pallas_call(
        _stats_body,
        grid=(_B, _NB),
        in_specs=[
            pl.BlockSpec((1, _BLKN, _D), lambda b, nb: (b, nb, 0)),
            pl.BlockSpec((_D, _E), lambda b, nb: (0, 0)),
        ],
        out_specs=[
            pl.BlockSpec((1, 8, _BLKN), lambda b, nb: (b, 0, nb)),
            pl.BlockSpec((1, 1), lambda b, nb: (0, 0)),
            pl.BlockSpec((1, 1), lambda b, nb: (0, 0)),
        ],
        out_shape=[
            jax.ShapeDtypeStruct((_B, 8, _N), jnp.float32),
            jax.ShapeDtypeStruct((1, 1), jnp.float32),
            jax.ShapeDtypeStruct((1, 1), jnp.float32),
        ],
        scratch_shapes=[
            pltpu.VMEM((_E, 1), jnp.float32),
            pltpu.VMEM((_E, 1), jnp.float32),
            pltpu.VMEM((_E, 1), jnp.float32),
            pltpu.VMEM((1, 1), jnp.float32),
        ],
        compiler_params=pltpu.CompilerParams(
            dimension_semantics=("arbitrary", "arbitrary"),
        ),
    )(token_inputs, w_gate)

    disp_t, comb_t = pl.pallas_call(
        _writer_body,
        grid=(_B, _EB),
        in_specs=[
            pl.BlockSpec(memory_space=pltpu.SMEM),
            pl.BlockSpec((1, 8, _N), lambda b, eb: (b, 0, 0)),
        ],
        out_specs=[
            pl.BlockSpec((1, _EC, _C, _N), lambda b, eb: (b, eb, 0, 0)),
            pl.BlockSpec((1, _EC, _C, _N), lambda b, eb: (b, eb, 0, 0)),
        ],
        out_shape=[
            jax.ShapeDtypeStruct((_B, _E, _C, _N), jnp.float32),
            jax.ShapeDtypeStruct((_B, _E, _C, _N), jnp.float32),
        ],
        compiler_params=pltpu.CompilerParams(
            dimension_semantics=("arbitrary", "arbitrary"),
        ),
    )(cap, meta)

    disp = jnp.transpose(disp_t, (0, 3, 1, 2))
    comb = jnp.transpose(comb_t, (0, 3, 1, 2))
    return disp, comb, aux[0, 0], zloss[0, 0]


# speedup vs baseline: 1.0215x; 1.0215x over previous
import jax
import jax.numpy as jnp
from jax.experimental import pallas as pl
from jax.experimental.pallas import tpu as pltpu

_B, _N, _D, _E, _C = 4, 2048, 4096, 64, 64
_BLKN = 512
_NB = _N // _BLKN


def _router_body(cap_ref, x_ref, w_ref, disp_ref, comb_ref, aux_ref, zloss_ref,
                 base_ref, counts_ref, dens_ref, zsq_ref):
    b = pl.program_id(0)
    nb = pl.program_id(1)

    @pl.when((b == 0) & (nb == 0))
    def _init_all():
        counts_ref[...] = jnp.zeros_like(counts_ref)
        dens_ref[...] = jnp.zeros_like(dens_ref)
        zsq_ref[...] = jnp.zeros_like(zsq_ref)

    @pl.when(nb == 0)
    def _init_batch():
        base_ref[...] = jnp.zeros_like(base_ref)

    x = x_ref[0]
    w = w_ref[...]
    lt = jax.lax.dot_general(w, x, (((0,), (1,)), ((), ())),
                             preferred_element_type=jnp.float32)
    m = jnp.max(lt, axis=0, keepdims=True)
    ex = jnp.exp(lt - m)
    s = jnp.sum(ex, axis=0, keepdims=True)
    z = m + jnp.log(s)
    zsq_ref[...] += jnp.sum(z * z).reshape(1, 1)
    gate = 1.0 / s
    dens_ref[...] += jnp.sum(ex / s, axis=1, keepdims=True)

    eids = jax.lax.broadcasted_iota(jnp.int32, (_E, _BLKN), 0)
    idx = jnp.min(jnp.where(lt == m, eids, _E), axis=0, keepdims=True)
    mask = (idx == eids).astype(jnp.float32)
    ri = jax.lax.broadcasted_iota(jnp.int32, (_BLKN, _BLKN), 0)
    ci = jax.lax.broadcasted_iota(jnp.int32, (_BLKN, _BLKN), 1)
    triu = (ri <= ci).astype(jnp.float32)
    cs = jnp.dot(mask, triu, preferred_element_type=jnp.float32)
    pos = base_ref[...] + cs
    tot = cs[:, _BLKN - 1:_BLKN]
    base_ref[...] += tot
    counts_ref[...] += tot
    p = jnp.sum(pos * mask, axis=0, keepdims=True).astype(jnp.int32) - 1

    cap = cap_ref[0, 0]
    cids = jax.lax.broadcasted_iota(jnp.int32, (_C, _BLKN), 0)
    ocf = ((p == cids) & (cids < cap)).astype(jnp.float32)
    ocg = ocf * gate
    disp_ref[0] = mask[:, None, :] * ocf[None, :, :]
    comb_ref[0] = mask[:, None, :] * ocg[None, :, :]

    @pl.when((b == _B - 1) & (nb == _NB - 1))
    def _fin():
        scale = 1.0 / (_B * _N)
        dens = dens_ref[...] * scale
        proxy = counts_ref[...] * scale
        aux_ref[...] = (jnp.sum(dens * proxy) * _E).reshape(1, 1)
        zloss_ref[...] = zsq_ref[...] * scale


def kernel(token_inputs, w_gate, expert_capacity):
    cap = jnp.asarray(expert_capacity, jnp.int32).reshape(1, 1)
    disp_t, comb_t, aux, zloss = pl.pallas_call(
        _router_body,
        grid=(_B, _NB),
        in_specs=[
            pl.BlockSpec(memory_space=pltpu.SMEM),
            pl.BlockSpec((1, _BLKN, _D), lambda b, nb: (b, nb, 0)),
            pl.BlockSpec((_D, _E), lambda b, nb: (0, 0)),
        ],
        out_specs=[
            pl.BlockSpec((1, _E, _C, _BLKN), lambda b, nb: (b, 0, 0, nb)),
            pl.BlockSpec((1, _E, _C, _BLKN), lambda b, nb: (b, 0, 0, nb)),
            pl.BlockSpec((1, 1), lambda b, nb: (0, 0)),
            pl.BlockSpec((1, 1), lambda b, nb: (0, 0)),
        ],
        out_shape=[
            jax.ShapeDtypeStruct((_B, _E, _C, _N), jnp.float32),
            jax.ShapeDtypeStruct((_B, _E, _C, _N), jnp.float32),
            jax.ShapeDtypeStruct((1, 1), jnp.float32),
            jax.ShapeDtypeStruct((1, 1), jnp.float32),
        ],
        scratch_shapes=[
            pltpu.VMEM((_E, 1), jnp.float32),
            pltpu.VMEM((_E, 1), jnp.float32),
            pltpu.VMEM((_E, 1), jnp.float32),
            pltpu.VMEM((1, 1), jnp.float32),
        ],
        compiler_params=pltpu.CompilerParams(
            dimension_semantics=("arbitrary", "arbitrary"),
        ),
    )(cap, token_inputs, w_gate)
    disp = jnp.transpose(disp_t, (0, 3, 1, 2))
    comb = jnp.transpose(comb_t, (0, 3, 1, 2))
    return disp, comb, aux[0, 0], zloss[0, 0]
